# baseline (speedup 1.0000x reference)
import math
import functools

import jax
import jax.numpy as jnp
from jax import lax
from jax.experimental import pallas as pl
from jax.experimental.pallas import tpu as pltpu
from jax.experimental.pallas import tpu_sc as plsc

R, N = 128, 32768
K = math.floor(0.1 * N)
NVEC = N // 16
CAP = 4096
NW = 32
ROWS_PER_W = R // NW
SENT_HI = 0x7FFFFFFF


def _lane0(v):
    return lax.squeeze(lax.slice(v, (0,), (1,)), (0,))


def _lane15(v):
    return lax.squeeze(lax.slice(v, (15,), (16,)), (0,))


def _sc_body(x_hbm, out_hbm, P0, P1, h1a, h1b, h2a, h2b,
             BA0, BB0, TA0, TB0, IA0, IB0,
             BA1, BB1, TA1, TB1, IA1, IB1):
    wid = lax.axis_index("s") * 2 + lax.axis_index("c")
    lane = lax.iota(jnp.int32, 16)
    zeros16 = jnp.zeros((16,), jnp.int32)

    czero, _ = plsc.scan_count(zeros16)
    c0 = jnp.min(czero)
    e0 = jnp.int32(1) - c0

    def hist_bump(href, d, cnt, lastm):
        plsc.addupdate_scatter(href, [d], cnt + e0, mask=lastm)

    def rank_bump(href, d, cnt, lastm):
        base = plsc.load_gather(href, [d])
        plsc.store_scatter(href, [d], base + cnt + e0, mask=lastm)
        return base

    def clear2(ha, hb, nv):
        def body(i, _):
            ha[pl.ds(i * 16, 16)] = zeros16
            hb[pl.ds(i * 16, 16)] = zeros16
            return 0
        lax.fori_loop(0, nv, body, 0)

    def radix_pass2(shift, s0, d0, is0, id0, nb0, s1, d1, is1, id1, nb1):
        clear2(h2a, h2b, 64)
        nb = jnp.maximum(nb0, nb1)

        def hist(i, _):
            for (src, href, nbx, sl) in ((s0, h2a, nb0, 0), (s0, h2a, nb0, 1),
                                         (s1, h2b, nb1, 0), (s1, h2b, nb1, 1)):
                mv = (zeros16 + 2 * i + sl) < nbx
                u = src[pl.ds((2 * i + sl) * 16, 16)]
                g = (u >> shift) & 1023
                cv, lv = plsc.scan_count(g, mv)
                hist_bump(href, g, cv, lv)
            return 0
        lax.fori_loop(0, (nb + 1) >> 1, hist, 0)

        def csum(i, carry):
            ca, cb = carry
            va = h2a[pl.ds(i * 16, 16)]
            vb = h2b[pl.ds(i * 16, 16)]
            sa = plsc.cumsum(va)
            sb = plsc.cumsum(vb)
            h2a[pl.ds(i * 16, 16)] = sa - va + ca
            h2b[pl.ds(i * 16, 16)] = sb - vb + cb
            return (ca + _lane15(sa), cb + _lane15(sb))
        lax.fori_loop(0, 64, csum, (jnp.int32(0), jnp.int32(0)))

        def scat(i, _):
            vi = zeros16 + i
            m0 = vi < nb0
            m1 = vi < nb1
            u0 = s0[pl.ds(i * 16, 16)]
            u1 = s1[pl.ds(i * 16, 16)]
            g0 = (u0 >> shift) & 1023
            g1 = (u1 >> shift) & 1023
            c0v, l0v = plsc.scan_count(g0, m0)
            c1v, l1v = plsc.scan_count(g1, m1)
            b0 = rank_bump(h2a, g0, c0v, l0v)
            b1 = rank_bump(h2b, g1, c1v, l1v)
            o0 = b0 + c0v - c0
            o1 = b1 + c1v - c0
            plsc.store_scatter(d0, [o0], u0, mask=m0)
            plsc.store_scatter(d1, [o1], u1, mask=m1)
            if is0 is not None:
                plsc.store_scatter(id0, [o0], is0[pl.ds(i * 16, 16)], mask=m0)
                plsc.store_scatter(id1, [o1], is1[pl.ds(i * 16, 16)], mask=m1)
            return 0
        lax.fori_loop(0, nb, scat, 0)

    def pair_body(pp, _):
        row0 = wid * ROWS_PER_W + pp * 2
        row1 = row0 + 1
        pltpu.sync_copy(x_hbm.at[row0], P0)
        pltpu.sync_copy(x_hbm.at[row1], P1)

        clear2(h1a, h1b, 256)

        def mh(i, carry):
            acc0, acc1, acc2, acc3 = carry
            accs = []
            for (Pr, href, sl) in ((P0, h1a, 0), (P0, h1a, 16),
                                   (P1, h1b, 0), (P1, h1b, 16)):
                a = jnp.abs(Pr[pl.ds(i * 32 + sl, 16)])
                g = plsc.bitcast(a, jnp.int32) >> 19
                cv, lv = plsc.scan_count(g)
                hist_bump(href, g, cv, lv)
                accs.append(a)
            return (jnp.maximum(acc0, accs[0]), jnp.maximum(acc1, accs[1]),
                    jnp.maximum(acc2, accs[2]), jnp.maximum(acc3, accs[3]))
        z16f = jnp.zeros((16,), jnp.float32)
        acc0, acc1, acc2, acc3 = lax.fori_loop(
            0, NVEC // 2, mh, (z16f, z16f, z16f, z16f))
        m0 = jnp.max(jnp.maximum(acc0, acc1))
        m1 = jnp.max(jnp.maximum(acc2, acc3))
        r0 = jnp.float32(1.0) / (jnp.zeros((16,), jnp.float32) + m0)
        r1 = jnp.float32(1.0) / (jnp.zeros((16,), jnp.float32) + m1)

        def cs1(i, carry):
            ca, t1a, t2a, cb, t1b, t2b = carry
            va = h1a[pl.ds(i * 16, 16)]
            vb = h1b[pl.ds(i * 16, 16)]
            sa = plsc.cumsum(va)
            sb = plsc.cumsum(vb)
            exa = sa - va + ca
            exb = sb - vb + cb
            t1a = t1a + _lane0(plsc.all_reduce_population_count(exa < K))
            t2a = t2a + _lane0(plsc.all_reduce_population_count(exa <= N - K))
            t1b = t1b + _lane0(plsc.all_reduce_population_count(exb < K))
            t2b = t2b + _lane0(plsc.all_reduce_population_count(exb <= N - K))
            return (ca + _lane15(sa), t1a, t2a, cb + _lane15(sb), t1b, t2b)
        z = jnp.int32(0)
        _, t1_0, t2_0, _, t1_1, t2_1 = lax.fori_loop(
            0, 256, cs1, (z, z, z, z, z, z))
        H0 = t2_0 - 1
        H1 = t2_1 - 1

        def cp(i, carry):
            pb0, pt0, pb1, pt1 = carry
            sl = pl.ds(i * 16, 16)
            a0 = jnp.abs(P0[sl])
            a1 = jnp.abs(P1[sl])
            g0 = plsc.bitcast(a0, jnp.int32) >> 19
            g1 = plsc.bitcast(a1, jnp.int32) >> 19
            p0 = a0 * r0
            p1 = a1 * r1
            P0[sl] = p0
            P1[sl] = p1
            u0 = plsc.bitcast(p0, jnp.int32)
            u1 = plsc.bitcast(p1, jnp.int32)
            mB0 = g0 < t1_0
            mT0 = g0 >= H0
            mB1 = g1 < t1_1
            mT1 = g1 >= H1
            ix = lane + i * 16
            plsc.store_compressed(BA0.at[pl.ds(pb0, 16)], u0, mask=mB0)
            plsc.store_compressed(TA0.at[pl.ds(pt0, 16)], u0, mask=mT0)
            plsc.store_compressed(IA0.at[pl.ds(pt0, 16)], ix, mask=mT0)
            plsc.store_compressed(BA1.at[pl.ds(pb1, 16)], u1, mask=mB1)
            plsc.store_compressed(TA1.at[pl.ds(pt1, 16)], u1, mask=mT1)
            plsc.store_compressed(IA1.at[pl.ds(pt1, 16)], ix, mask=mT1)
            pb0 = pb0 + _lane0(plsc.all_reduce_population_count(mB0))
            pt0 = pt0 + _lane0(plsc.all_reduce_population_count(mT0))
            pb1 = pb1 + _lane0(plsc.all_reduce_population_count(mB1))
            pt1 = pt1 + _lane0(plsc.all_reduce_population_count(mT1))
            return (pb0, pt0, pb1, pt1)
        pb0, pt0, pb1, pt1 = lax.fori_loop(0, NVEC, cp, (z, z, z, z))

        sent = jnp.full((16,), SENT_HI, jnp.int32)
        BA0[pl.ds(pb0, 16)] = sent
        BA1[pl.ds(pb1, 16)] = sent
        TA0[pl.ds(pt0, 16)] = zeros16
        TA1[pl.ds(pt1, 16)] = zeros16
        IA0[pl.ds(pt0, 16)] = zeros16
        IA1[pl.ds(pt1, 16)] = zeros16
        nbB0 = (pb0 + 15) >> 4
        nbB1 = (pb1 + 15) >> 4
        nbT0 = (pt0 + 15) >> 4
        nbT1 = (pt1 + 15) >> 4
        STp0 = nbT0 * 16
        STp1 = nbT1 * 16

        radix_pass2(10, BA0, BB0, None, None, nbB0, BA1, BB1, None, None, nbB1)
        radix_pass2(20, BB0, BA0, None, None, nbB0, BB1, BA1, None, None, nbB1)

        radix_pass2(0, TA0, TB0, IA0, IB0, nbT0, TA1, TB1, IA1, IB1, nbT1)
        radix_pass2(10, TB0, TA0, IB0, IA0, nbT0, TB1, TA1, IB1, IA1, nbT1)
        radix_pass2(20, TA0, TB0, IA0, IB0, nbT0, TA1, TB1, IA1, IB1, nbT1)

        def rep(i, _):
            t = jnp.minimum(lane + i * 16, K - 1)
            j0 = STp0 - K + t
            j1 = STp1 - K + t
            vu0 = plsc.load_gather(TB0, [j0])
            ti0 = plsc.load_gather(IB0, [j0])
            bu0 = plsc.load_gather(BA0, [K - 1 - t])
            vu1 = plsc.load_gather(TB1, [j1])
            ti1 = plsc.load_gather(IB1, [j1])
            bu1 = plsc.load_gather(BA1, [K - 1 - t])
            v0 = plsc.bitcast(vu0, jnp.float32)
            b0 = plsc.bitcast(bu0, jnp.float32)
            v1 = plsc.bitcast(vu1, jnp.float32)
            b1 = plsc.bitcast(bu1, jnp.float32)
            plsc.store_scatter(P0, [ti0], v0 - (v0 - b0))
            plsc.store_scatter(P1, [ti1], v1 - (v1 - b1))
            return 0
        lax.fori_loop(0, (K + 15) // 16, rep, 0)

        pltpu.sync_copy(P0, out_hbm.at[row0])
        pltpu.sync_copy(P1, out_hbm.at[row1])
        return 0

    lax.fori_loop(0, ROWS_PER_W // 2, pair_body, 0)


@functools.partial(jax.jit, static_argnums=())
def _sc_topk_replace(x):
    row_scratch = []
    for _ in range(2):
        row_scratch += [pltpu.VMEM((CAP,), jnp.int32) for _ in range(6)]
    kfn = pl.kernel(
        _sc_body,
        out_type=jax.ShapeDtypeStruct((R, N), jnp.float32),
        mesh=plsc.VectorSubcoreMesh(core_axis_name="c", subcore_axis_name="s"),
        compiler_params=pltpu.CompilerParams(needs_layout_passes=False),
        scratch_types=[
            pltpu.VMEM((N,), jnp.float32),
            pltpu.VMEM((N,), jnp.float32),
            pltpu.VMEM((4096,), jnp.int32),
            pltpu.VMEM((4096,), jnp.int32),
            pltpu.VMEM((1024,), jnp.int32),
            pltpu.VMEM((1024,), jnp.int32),
        ] + row_scratch,
    )
    return kfn(x)


def _rotl(v, d):
    u = jnp.uint32(d)
    return (v << u) | (v >> jnp.uint32(32 - d))


def _ugen_body(u_ref, *, block_cols):
    i = pl.program_id(0)
    rows_blk, cols_blk = R, block_cols
    row = lax.broadcasted_iota(jnp.uint32, (rows_blk, cols_blk), 0)
    col = lax.broadcasted_iota(jnp.uint32, (rows_blk, cols_blk), 1)
    n = row * jnp.uint32(N) + col + jnp.uint32(block_cols) * i.astype(jnp.uint32)
    ks0 = jnp.uint32(0)
    ks1 = jnp.uint32(42)
    ks2 = jnp.uint32(42 ^ 0x1BD11BDA)
    x0 = jnp.full_like(n, ks0)
    x1 = n + ks1

    def rounds(x0, x1, rots):
        for r in rots:
            x0 = x0 + x1
            x1 = _rotl(x1, r)
            x1 = x0 ^ x1
        return x0, x1

    ra = (13, 15, 26, 6)
    rb = (17, 29, 16, 24)
    x0, x1 = rounds(x0, x1, ra)
    x0 += ks1
    x1 += ks2 + jnp.uint32(1)
    x0, x1 = rounds(x0, x1, rb)
    x0 += ks2
    x1 += ks0 + jnp.uint32(2)
    x0, x1 = rounds(x0, x1, ra)
    x0 += ks0
    x1 += ks1 + jnp.uint32(3)
    x0, x1 = rounds(x0, x1, rb)
    x0 += ks1
    x1 += ks2 + jnp.uint32(4)
    x0, x1 = rounds(x0, x1, ra)
    x0 += ks2
    x1 += ks0 + jnp.uint32(5)
    bits = x0 ^ x1

    fb = (bits >> jnp.uint32(9)) | jnp.uint32(0x3F800000)
    u_ref[...] = lax.bitcast_convert_type(fb, jnp.float32) - jnp.float32(1.0)


def _gen_uniform():
    block_cols = 4096
    return pl.pallas_call(
        functools.partial(_ugen_body, block_cols=block_cols),
        grid=(N // block_cols,),
        in_specs=[],
        out_specs=pl.BlockSpec((R, block_cols), lambda i: (0, i)),
        out_shape=jax.ShapeDtypeStruct((R, N), jnp.float32),
    )()


def _sel_body(x_ref, p_ref, u_ref, o_ref):
    x = x_ref[...]
    keep = u_ref[...] < (jnp.float32(1.0) - p_ref[...])
    o_ref[...] = jnp.where(keep, x, jnp.float32(0.0))


def _apply_mask(x, prob, u):
    block_cols = 8192
    return pl.pallas_call(
        _sel_body,
        grid=(N // block_cols,),
        in_specs=[
            pl.BlockSpec((R, block_cols), lambda i: (0, i)),
            pl.BlockSpec((R, block_cols), lambda i: (0, i)),
            pl.BlockSpec((R, block_cols), lambda i: (0, i)),
        ],
        out_specs=pl.BlockSpec((R, block_cols), lambda i: (0, i)),
        out_shape=jax.ShapeDtypeStruct((R, N), jnp.float32),
    )(x, prob, u)


def kernel(x):
    u = _gen_uniform()
    new_prob = _sc_topk_replace(x)
    return _apply_mask(x, new_prob, u)


# speedup vs baseline: 1.3378x; 1.3378x over previous
import math
import functools

import jax
import jax.numpy as jnp
from jax import lax
from jax.experimental import pallas as pl
from jax.experimental.pallas import tpu as pltpu
from jax.experimental.pallas import tpu_sc as plsc

R, N = 128, 32768
K = math.floor(0.1 * N)
NVEC = N // 16
CAP = 4096
NW = 32
ROWS_PER_W = R // NW
SENT_HI = 0x7FFFFFFF


def _lane0(v):
    return lax.squeeze(lax.slice(v, (0,), (1,)), (0,))


def _lane15(v):
    return lax.squeeze(lax.slice(v, (15,), (16,)), (0,))


def _sc_body(x_hbm, out_hbm, P0, P1, h1a, h1b, h2a, h2b,
             BA0, BB0, TA0, TB0, IA0, IB0,
             BA1, BB1, TA1, TB1, IA1, IB1):
    wid = lax.axis_index("s") * 2 + lax.axis_index("c")
    lane = lax.iota(jnp.int32, 16)
    zeros16 = jnp.zeros((16,), jnp.int32)

    czero, _ = plsc.scan_count(zeros16)
    c0 = jnp.min(czero)
    e0 = jnp.int32(1) - c0

    def hist_bump(href, d, cnt, lastm):
        plsc.addupdate_scatter(href, [d], cnt + e0, mask=lastm)

    def rank_bump(href, d, cnt, lastm):
        base = plsc.load_gather(href, [d])
        plsc.store_scatter(href, [d], base + cnt + e0, mask=lastm)
        return base

    def clear2(ha, hb, nv):
        def body(i, _):
            ha[pl.ds(i * 16, 16)] = zeros16
            hb[pl.ds(i * 16, 16)] = zeros16
            return 0
        lax.fori_loop(0, nv, body, 0)

    def radix_pass2(shift, s0, d0, is0, id0, nb0, s1, d1, is1, id1, nb1):
        clear2(h2a, h2b, 64)
        nb = jnp.maximum(nb0, nb1)

        def hist(i, _):
            vi = zeros16 + i
            m0 = vi < nb0
            m1 = vi < nb1
            u0 = s0[pl.ds(i * 16, 16)]
            u1 = s1[pl.ds(i * 16, 16)]
            g0 = (u0 >> shift) & 1023
            g1 = (u1 >> shift) & 1023
            c0v, l0v = plsc.scan_count(g0, m0)
            c1v, l1v = plsc.scan_count(g1, m1)
            hist_bump(h2a, g0, c0v, l0v)
            hist_bump(h2b, g1, c1v, l1v)
            return 0
        lax.fori_loop(0, nb, hist, 0)

        def csum(i, carry):
            ca, cb = carry
            va = h2a[pl.ds(i * 16, 16)]
            vb = h2b[pl.ds(i * 16, 16)]
            sa = plsc.cumsum(va)
            sb = plsc.cumsum(vb)
            h2a[pl.ds(i * 16, 16)] = sa - va + ca
            h2b[pl.ds(i * 16, 16)] = sb - vb + cb
            return (ca + _lane15(sa), cb + _lane15(sb))
        lax.fori_loop(0, 64, csum, (jnp.int32(0), jnp.int32(0)))

        def scat(i, _):
            vi = zeros16 + i
            m0 = vi < nb0
            m1 = vi < nb1
            u0 = s0[pl.ds(i * 16, 16)]
            u1 = s1[pl.ds(i * 16, 16)]
            g0 = (u0 >> shift) & 1023
            g1 = (u1 >> shift) & 1023
            c0v, l0v = plsc.scan_count(g0, m0)
            c1v, l1v = plsc.scan_count(g1, m1)
            b0 = rank_bump(h2a, g0, c0v, l0v)
            b1 = rank_bump(h2b, g1, c1v, l1v)
            o0 = b0 + c0v - c0
            o1 = b1 + c1v - c0
            plsc.store_scatter(d0, [o0], u0, mask=m0)
            plsc.store_scatter(d1, [o1], u1, mask=m1)
            if is0 is not None:
                plsc.store_scatter(id0, [o0], is0[pl.ds(i * 16, 16)], mask=m0)
                plsc.store_scatter(id1, [o1], is1[pl.ds(i * 16, 16)], mask=m1)
            return 0
        lax.fori_loop(0, nb, scat, 0)

    def pair_body(pp, _):
        row0 = wid * ROWS_PER_W + pp * 2
        row1 = row0 + 1
        pltpu.sync_copy(x_hbm.at[row0], P0)
        pltpu.sync_copy(x_hbm.at[row1], P1)

        clear2(h1a, h1b, 256)

        def mh(i, carry):
            acc0, acc1 = carry
            a0 = jnp.abs(P0[pl.ds(i * 16, 16)])
            a1 = jnp.abs(P1[pl.ds(i * 16, 16)])
            g0 = plsc.bitcast(a0, jnp.int32) >> 19
            g1 = plsc.bitcast(a1, jnp.int32) >> 19
            c0v, l0v = plsc.scan_count(g0)
            c1v, l1v = plsc.scan_count(g1)
            hist_bump(h1a, g0, c0v, l0v)
            hist_bump(h1b, g1, c1v, l1v)
            return (jnp.maximum(acc0, a0), jnp.maximum(acc1, a1))
        z16f = jnp.zeros((16,), jnp.float32)
        acc0, acc1 = lax.fori_loop(0, NVEC, mh, (z16f, z16f))
        m0 = jnp.max(acc0)
        m1 = jnp.max(acc1)
        r0 = jnp.float32(1.0) / (jnp.zeros((16,), jnp.float32) + m0)
        r1 = jnp.float32(1.0) / (jnp.zeros((16,), jnp.float32) + m1)

        def cs1(i, carry):
            ca, t1a, t2a, cb, t1b, t2b = carry
            va = h1a[pl.ds(i * 16, 16)]
            vb = h1b[pl.ds(i * 16, 16)]
            sa = plsc.cumsum(va)
            sb = plsc.cumsum(vb)
            exa = sa - va + ca
            exb = sb - vb + cb
            t1a = t1a + _lane0(plsc.all_reduce_population_count(exa < K))
            t2a = t2a + _lane0(plsc.all_reduce_population_count(exa <= N - K))
            t1b = t1b + _lane0(plsc.all_reduce_population_count(exb < K))
            t2b = t2b + _lane0(plsc.all_reduce_population_count(exb <= N - K))
            return (ca + _lane15(sa), t1a, t2a, cb + _lane15(sb), t1b, t2b)
        z = jnp.int32(0)
        _, t1_0, t2_0, _, t1_1, t2_1 = lax.fori_loop(
            0, 256, cs1, (z, z, z, z, z, z))
        H0 = t2_0 - 1
        H1 = t2_1 - 1

        def cp(i, carry):
            pb0, pt0, pb1, pt1 = carry
            sl = pl.ds(i * 16, 16)
            a0 = jnp.abs(P0[sl])
            a1 = jnp.abs(P1[sl])
            g0 = plsc.bitcast(a0, jnp.int32) >> 19
            g1 = plsc.bitcast(a1, jnp.int32) >> 19
            p0 = a0 * r0
            p1 = a1 * r1
            P0[sl] = p0
            P1[sl] = p1
            u0 = plsc.bitcast(p0, jnp.int32)
            u1 = plsc.bitcast(p1, jnp.int32)
            mB0 = g0 < t1_0
            mT0 = g0 >= H0
            mB1 = g1 < t1_1
            mT1 = g1 >= H1
            ix = lane + i * 16
            plsc.store_compressed(BA0.at[pl.ds(pb0, 16)], u0, mask=mB0)
            plsc.store_compressed(TA0.at[pl.ds(pt0, 16)], u0, mask=mT0)
            plsc.store_compressed(IA0.at[pl.ds(pt0, 16)], ix, mask=mT0)
            plsc.store_compressed(BA1.at[pl.ds(pb1, 16)], u1, mask=mB1)
            plsc.store_compressed(TA1.at[pl.ds(pt1, 16)], u1, mask=mT1)
            plsc.store_compressed(IA1.at[pl.ds(pt1, 16)], ix, mask=mT1)
            pb0 = pb0 + _lane0(plsc.all_reduce_population_count(mB0))
            pt0 = pt0 + _lane0(plsc.all_reduce_population_count(mT0))
            pb1 = pb1 + _lane0(plsc.all_reduce_population_count(mB1))
            pt1 = pt1 + _lane0(plsc.all_reduce_population_count(mT1))
            return (pb0, pt0, pb1, pt1)
        pb0, pt0, pb1, pt1 = lax.fori_loop(0, NVEC, cp, (z, z, z, z))

        sent = jnp.full((16,), SENT_HI, jnp.int32)
        BA0[pl.ds(pb0, 16)] = sent
        BA1[pl.ds(pb1, 16)] = sent
        TA0[pl.ds(pt0, 16)] = zeros16
        TA1[pl.ds(pt1, 16)] = zeros16
        IA0[pl.ds(pt0, 16)] = zeros16
        IA1[pl.ds(pt1, 16)] = zeros16
        nbB0 = (pb0 + 15) >> 4
        nbB1 = (pb1 + 15) >> 4
        nbT0 = (pt0 + 15) >> 4
        nbT1 = (pt1 + 15) >> 4
        STp0 = nbT0 * 16
        STp1 = nbT1 * 16

        radix_pass2(10, BA0, BB0, None, None, nbB0, BA1, BB1, None, None, nbB1)
        radix_pass2(20, BB0, BA0, None, None, nbB0, BB1, BA1, None, None, nbB1)

        radix_pass2(0, TA0, TB0, IA0, IB0, nbT0, TA1, TB1, IA1, IB1, nbT1)
        radix_pass2(10, TB0, TA0, IB0, IA0, nbT0, TB1, TA1, IB1, IA1, nbT1)
        radix_pass2(20, TA0, TB0, IA0, IB0, nbT0, TA1, TB1, IA1, IB1, nbT1)

        def rep(i, _):
            t = jnp.minimum(lane + i * 16, K - 1)
            j0 = STp0 - K + t
            j1 = STp1 - K + t
            vu0 = plsc.load_gather(TB0, [j0])
            ti0 = plsc.load_gather(IB0, [j0])
            bu0 = plsc.load_gather(BA0, [K - 1 - t])
            vu1 = plsc.load_gather(TB1, [j1])
            ti1 = plsc.load_gather(IB1, [j1])
            bu1 = plsc.load_gather(BA1, [K - 1 - t])
            v0 = plsc.bitcast(vu0, jnp.float32)
            b0 = plsc.bitcast(bu0, jnp.float32)
            v1 = plsc.bitcast(vu1, jnp.float32)
            b1 = plsc.bitcast(bu1, jnp.float32)
            plsc.store_scatter(P0, [ti0], v0 - (v0 - b0))
            plsc.store_scatter(P1, [ti1], v1 - (v1 - b1))
            return 0
        lax.fori_loop(0, (K + 15) // 16, rep, 0)

        pltpu.sync_copy(P0, out_hbm.at[row0])
        pltpu.sync_copy(P1, out_hbm.at[row1])
        return 0

    lax.fori_loop(0, ROWS_PER_W // 2, pair_body, 0)


@functools.partial(jax.jit, static_argnums=())
def _sc_topk_replace(x):
    row_scratch = []
    for _ in range(2):
        row_scratch += [pltpu.VMEM((CAP,), jnp.int32) for _ in range(6)]
    kfn = pl.kernel(
        _sc_body,
        out_type=jax.ShapeDtypeStruct((R, N), jnp.float32),
        mesh=plsc.VectorSubcoreMesh(core_axis_name="c", subcore_axis_name="s"),
        compiler_params=pltpu.CompilerParams(needs_layout_passes=False),
        scratch_types=[
            pltpu.VMEM((N,), jnp.float32),
            pltpu.VMEM((N,), jnp.float32),
            pltpu.VMEM((4096,), jnp.int32),
            pltpu.VMEM((4096,), jnp.int32),
            pltpu.VMEM((1024,), jnp.int32),
            pltpu.VMEM((1024,), jnp.int32),
        ] + row_scratch,
    )
    return kfn(x)


def _rotl(v, d):
    u = jnp.uint32(d)
    return (v << u) | (v >> jnp.uint32(32 - d))


def _ugen_body(u_ref, *, block_cols):
    i = pl.program_id(0)
    rows_blk, cols_blk = R, block_cols
    row = lax.broadcasted_iota(jnp.uint32, (rows_blk, cols_blk), 0)
    col = lax.broadcasted_iota(jnp.uint32, (rows_blk, cols_blk), 1)
    n = row * jnp.uint32(N) + col + jnp.uint32(block_cols) * i.astype(jnp.uint32)
    ks0 = jnp.uint32(0)
    ks1 = jnp.uint32(42)
    ks2 = jnp.uint32(42 ^ 0x1BD11BDA)
    x0 = jnp.full_like(n, ks0)
    x1 = n + ks1

    def rounds(x0, x1, rots):
        for r in rots:
            x0 = x0 + x1
            x1 = _rotl(x1, r)
            x1 = x0 ^ x1
        return x0, x1

    ra = (13, 15, 26, 6)
    rb = (17, 29, 16, 24)
    x0, x1 = rounds(x0, x1, ra)
    x0 += ks1
    x1 += ks2 + jnp.uint32(1)
    x0, x1 = rounds(x0, x1, rb)
    x0 += ks2
    x1 += ks0 + jnp.uint32(2)
    x0, x1 = rounds(x0, x1, ra)
    x0 += ks0
    x1 += ks1 + jnp.uint32(3)
    x0, x1 = rounds(x0, x1, rb)
    x0 += ks1
    x1 += ks2 + jnp.uint32(4)
    x0, x1 = rounds(x0, x1, ra)
    x0 += ks2
    x1 += ks0 + jnp.uint32(5)
    bits = x0 ^ x1

    fb = (bits >> jnp.uint32(9)) | jnp.uint32(0x3F800000)
    u_ref[...] = lax.bitcast_convert_type(fb, jnp.float32) - jnp.float32(1.0)


def _gen_uniform():
    block_cols = 4096
    return pl.pallas_call(
        functools.partial(_ugen_body, block_cols=block_cols),
        grid=(N // block_cols,),
        in_specs=[],
        out_specs=pl.BlockSpec((R, block_cols), lambda i: (0, i)),
        out_shape=jax.ShapeDtypeStruct((R, N), jnp.float32),
    )()


def _sel_body(x_ref, p_ref, u_ref, o_ref):
    x = x_ref[...]
    keep = u_ref[...] < (jnp.float32(1.0) - p_ref[...])
    o_ref[...] = jnp.where(keep, x, jnp.float32(0.0))


def _apply_mask(x, prob, u):
    block_cols = 8192
    return pl.pallas_call(
        _sel_body,
        grid=(N // block_cols,),
        in_specs=[
            pl.BlockSpec((R, block_cols), lambda i: (0, i)),
            pl.BlockSpec((R, block_cols), lambda i: (0, i)),
            pl.BlockSpec((R, block_cols), lambda i: (0, i)),
        ],
        out_specs=pl.BlockSpec((R, block_cols), lambda i: (0, i)),
        out_shape=jax.ShapeDtypeStruct((R, N), jnp.float32),
    )(x, prob, u)


def kernel(x):
    u = _gen_uniform()
    new_prob = _sc_topk_replace(x)
    return _apply_mask(x, new_prob, u)


# speedup vs baseline: 1.6311x; 1.2193x over previous
import math
import functools

import jax
import jax.numpy as jnp
from jax import lax
from jax.experimental import pallas as pl
from jax.experimental.pallas import tpu as pltpu
from jax.experimental.pallas import tpu_sc as plsc

R, N = 128, 32768
K = math.floor(0.1 * N)
NVEC = N // 16
CAP = 4096
NW = 32
ROWS_PER_W = R // NW
SENT_HI = 0x7FFFFFFF


def _lane0(v):
    return lax.squeeze(lax.slice(v, (0,), (1,)), (0,))


def _lane15(v):
    return lax.squeeze(lax.slice(v, (15,), (16,)), (0,))


def _sc_body(x_hbm, out_hbm, P0, P1, h1a, h1b, h2a, h2b,
             BA0, BB0, TA0, TB0, IA0, IB0,
             BA1, BB1, TA1, TB1, IA1, IB1):
    wid = lax.axis_index("s") * 2 + lax.axis_index("c")
    lane = lax.iota(jnp.int32, 16)
    zeros16 = jnp.zeros((16,), jnp.int32)

    czero, _ = plsc.scan_count(zeros16)
    c0 = jnp.min(czero)
    e0 = jnp.int32(1) - c0

    ones16 = jnp.ones((16,), jnp.int32)

    def hist_bump(href, d):
        plsc.addupdate_scatter(href, [d], ones16)

    def rank_bump(href, d, cnt, lastm):
        base = plsc.load_gather(href, [d])
        plsc.store_scatter(href, [d], base + cnt + e0, mask=lastm)
        return base

    def clear2(ha, hb, nv):
        def body(i, _):
            ha[pl.ds(i * 16, 16)] = zeros16
            hb[pl.ds(i * 16, 16)] = zeros16
            return 0
        lax.fori_loop(0, nv, body, 0)

    def radix_pass2(shift, s0, d0, is0, id0, nb0, s1, d1, is1, id1, nb1):
        clear2(h2a, h2b, 64)
        nb = jnp.maximum(nb0, nb1)

        def hist(i, _):
            vi = zeros16 + i
            m0 = vi < nb0
            m1 = vi < nb1
            u0 = s0[pl.ds(i * 16, 16)]
            u1 = s1[pl.ds(i * 16, 16)]
            g0 = (u0 >> shift) & 1023
            g1 = (u1 >> shift) & 1023
            plsc.addupdate_scatter(h2a, [g0], ones16, mask=m0)
            plsc.addupdate_scatter(h2b, [g1], ones16, mask=m1)
            return 0
        lax.fori_loop(0, nb, hist, 0)

        def csum(i, carry):
            ca, cb = carry
            va = h2a[pl.ds(i * 16, 16)]
            vb = h2b[pl.ds(i * 16, 16)]
            sa = plsc.cumsum(va)
            sb = plsc.cumsum(vb)
            h2a[pl.ds(i * 16, 16)] = sa - va + ca
            h2b[pl.ds(i * 16, 16)] = sb - vb + cb
            return (ca + _lane15(sa), cb + _lane15(sb))
        lax.fori_loop(0, 64, csum, (jnp.int32(0), jnp.int32(0)))

        def scat(i, _):
            vi = zeros16 + i
            m0 = vi < nb0
            m1 = vi < nb1
            u0 = s0[pl.ds(i * 16, 16)]
            u1 = s1[pl.ds(i * 16, 16)]
            g0 = (u0 >> shift) & 1023
            g1 = (u1 >> shift) & 1023
            c0v, l0v = plsc.scan_count(g0, m0)
            c1v, l1v = plsc.scan_count(g1, m1)
            b0 = rank_bump(h2a, g0, c0v, l0v)
            b1 = rank_bump(h2b, g1, c1v, l1v)
            o0 = b0 + c0v - c0
            o1 = b1 + c1v - c0
            plsc.store_scatter(d0, [o0], u0, mask=m0)
            plsc.store_scatter(d1, [o1], u1, mask=m1)
            if is0 is not None:
                plsc.store_scatter(id0, [o0], is0[pl.ds(i * 16, 16)], mask=m0)
                plsc.store_scatter(id1, [o1], is1[pl.ds(i * 16, 16)], mask=m1)
            return 0
        lax.fori_loop(0, nb, scat, 0)

    def pair_body(pp, _):
        row0 = wid * ROWS_PER_W + pp * 2
        row1 = row0 + 1
        pltpu.sync_copy(x_hbm.at[row0], P0)
        pltpu.sync_copy(x_hbm.at[row1], P1)

        clear2(h1a, h1b, 256)

        def mh(i, carry):
            acc0, acc1 = carry
            a0 = jnp.abs(P0[pl.ds(i * 16, 16)])
            a1 = jnp.abs(P1[pl.ds(i * 16, 16)])
            g0 = plsc.bitcast(a0, jnp.int32) >> 19
            g1 = plsc.bitcast(a1, jnp.int32) >> 19
            hist_bump(h1a, g0)
            hist_bump(h1b, g1)
            return (jnp.maximum(acc0, a0), jnp.maximum(acc1, a1))
        z16f = jnp.zeros((16,), jnp.float32)
        acc0, acc1 = lax.fori_loop(0, NVEC, mh, (z16f, z16f))
        m0 = jnp.max(acc0)
        m1 = jnp.max(acc1)
        r0 = jnp.float32(1.0) / (jnp.zeros((16,), jnp.float32) + m0)
        r1 = jnp.float32(1.0) / (jnp.zeros((16,), jnp.float32) + m1)

        def cs1(i, carry):
            ca, t1a, t2a, cb, t1b, t2b = carry
            va = h1a[pl.ds(i * 16, 16)]
            vb = h1b[pl.ds(i * 16, 16)]
            sa = plsc.cumsum(va)
            sb = plsc.cumsum(vb)
            exa = sa - va + ca
            exb = sb - vb + cb
            t1a = t1a + _lane0(plsc.all_reduce_population_count(exa < K))
            t2a = t2a + _lane0(plsc.all_reduce_population_count(exa <= N - K))
            t1b = t1b + _lane0(plsc.all_reduce_population_count(exb < K))
            t2b = t2b + _lane0(plsc.all_reduce_population_count(exb <= N - K))
            return (ca + _lane15(sa), t1a, t2a, cb + _lane15(sb), t1b, t2b)
        z = jnp.int32(0)
        _, t1_0, t2_0, _, t1_1, t2_1 = lax.fori_loop(
            0, 256, cs1, (z, z, z, z, z, z))
        H0 = t2_0 - 1
        H1 = t2_1 - 1

        def cp(i, carry):
            pb0, pt0, pb1, pt1 = carry
            sl = pl.ds(i * 16, 16)
            a0 = jnp.abs(P0[sl])
            a1 = jnp.abs(P1[sl])
            g0 = plsc.bitcast(a0, jnp.int32) >> 19
            g1 = plsc.bitcast(a1, jnp.int32) >> 19
            p0 = a0 * r0
            p1 = a1 * r1
            P0[sl] = p0
            P1[sl] = p1
            u0 = plsc.bitcast(p0, jnp.int32)
            u1 = plsc.bitcast(p1, jnp.int32)
            mB0 = g0 < t1_0
            mT0 = g0 >= H0
            mB1 = g1 < t1_1
            mT1 = g1 >= H1
            ix = lane + i * 16
            plsc.store_compressed(BA0.at[pl.ds(pb0, 16)], u0, mask=mB0)
            plsc.store_compressed(TA0.at[pl.ds(pt0, 16)], u0, mask=mT0)
            plsc.store_compressed(IA0.at[pl.ds(pt0, 16)], ix, mask=mT0)
            plsc.store_compressed(BA1.at[pl.ds(pb1, 16)], u1, mask=mB1)
            plsc.store_compressed(TA1.at[pl.ds(pt1, 16)], u1, mask=mT1)
            plsc.store_compressed(IA1.at[pl.ds(pt1, 16)], ix, mask=mT1)
            pb0 = pb0 + _lane0(plsc.all_reduce_population_count(mB0))
            pt0 = pt0 + _lane0(plsc.all_reduce_population_count(mT0))
            pb1 = pb1 + _lane0(plsc.all_reduce_population_count(mB1))
            pt1 = pt1 + _lane0(plsc.all_reduce_population_count(mT1))
            return (pb0, pt0, pb1, pt1)
        pb0, pt0, pb1, pt1 = lax.fori_loop(0, NVEC, cp, (z, z, z, z))

        sent = jnp.full((16,), SENT_HI, jnp.int32)
        BA0[pl.ds(pb0, 16)] = sent
        BA1[pl.ds(pb1, 16)] = sent
        TA0[pl.ds(pt0, 16)] = zeros16
        TA1[pl.ds(pt1, 16)] = zeros16
        IA0[pl.ds(pt0, 16)] = zeros16
        IA1[pl.ds(pt1, 16)] = zeros16
        nbB0 = (pb0 + 15) >> 4
        nbB1 = (pb1 + 15) >> 4
        nbT0 = (pt0 + 15) >> 4
        nbT1 = (pt1 + 15) >> 4
        STp0 = nbT0 * 16
        STp1 = nbT1 * 16

        radix_pass2(10, BA0, BB0, None, None, nbB0, BA1, BB1, None, None, nbB1)
        radix_pass2(20, BB0, BA0, None, None, nbB0, BB1, BA1, None, None, nbB1)

        radix_pass2(0, TA0, TB0, IA0, IB0, nbT0, TA1, TB1, IA1, IB1, nbT1)
        radix_pass2(10, TB0, TA0, IB0, IA0, nbT0, TB1, TA1, IB1, IA1, nbT1)
        radix_pass2(20, TA0, TB0, IA0, IB0, nbT0, TA1, TB1, IA1, IB1, nbT1)

        def rep(i, _):
            t = jnp.minimum(lane + i * 16, K - 1)
            j0 = STp0 - K + t
            j1 = STp1 - K + t
            vu0 = plsc.load_gather(TB0, [j0])
            ti0 = plsc.load_gather(IB0, [j0])
            bu0 = plsc.load_gather(BA0, [K - 1 - t])
            vu1 = plsc.load_gather(TB1, [j1])
            ti1 = plsc.load_gather(IB1, [j1])
            bu1 = plsc.load_gather(BA1, [K - 1 - t])
            v0 = plsc.bitcast(vu0, jnp.float32)
            b0 = plsc.bitcast(bu0, jnp.float32)
            v1 = plsc.bitcast(vu1, jnp.float32)
            b1 = plsc.bitcast(bu1, jnp.float32)
            plsc.store_scatter(P0, [ti0], v0 - (v0 - b0))
            plsc.store_scatter(P1, [ti1], v1 - (v1 - b1))
            return 0
        lax.fori_loop(0, (K + 15) // 16, rep, 0)

        pltpu.sync_copy(P0, out_hbm.at[row0])
        pltpu.sync_copy(P1, out_hbm.at[row1])
        return 0

    lax.fori_loop(0, ROWS_PER_W // 2, pair_body, 0)


@functools.partial(jax.jit, static_argnums=())
def _sc_topk_replace(x):
    row_scratch = []
    for _ in range(2):
        row_scratch += [pltpu.VMEM((CAP,), jnp.int32) for _ in range(6)]
    kfn = pl.kernel(
        _sc_body,
        out_type=jax.ShapeDtypeStruct((R, N), jnp.float32),
        mesh=plsc.VectorSubcoreMesh(core_axis_name="c", subcore_axis_name="s"),
        compiler_params=pltpu.CompilerParams(needs_layout_passes=False),
        scratch_types=[
            pltpu.VMEM((N,), jnp.float32),
            pltpu.VMEM((N,), jnp.float32),
            pltpu.VMEM((4096,), jnp.int32),
            pltpu.VMEM((4096,), jnp.int32),
            pltpu.VMEM((1024,), jnp.int32),
            pltpu.VMEM((1024,), jnp.int32),
        ] + row_scratch,
    )
    return kfn(x)


def _rotl(v, d):
    u = jnp.uint32(d)
    return (v << u) | (v >> jnp.uint32(32 - d))


def _ugen_body(u_ref, *, block_cols):
    i = pl.program_id(0)
    rows_blk, cols_blk = R, block_cols
    row = lax.broadcasted_iota(jnp.uint32, (rows_blk, cols_blk), 0)
    col = lax.broadcasted_iota(jnp.uint32, (rows_blk, cols_blk), 1)
    n = row * jnp.uint32(N) + col + jnp.uint32(block_cols) * i.astype(jnp.uint32)
    ks0 = jnp.uint32(0)
    ks1 = jnp.uint32(42)
    ks2 = jnp.uint32(42 ^ 0x1BD11BDA)
    x0 = jnp.full_like(n, ks0)
    x1 = n + ks1

    def rounds(x0, x1, rots):
        for r in rots:
            x0 = x0 + x1
            x1 = _rotl(x1, r)
            x1 = x0 ^ x1
        return x0, x1

    ra = (13, 15, 26, 6)
    rb = (17, 29, 16, 24)
    x0, x1 = rounds(x0, x1, ra)
    x0 += ks1
    x1 += ks2 + jnp.uint32(1)
    x0, x1 = rounds(x0, x1, rb)
    x0 += ks2
    x1 += ks0 + jnp.uint32(2)
    x0, x1 = rounds(x0, x1, ra)
    x0 += ks0
    x1 += ks1 + jnp.uint32(3)
    x0, x1 = rounds(x0, x1, rb)
    x0 += ks1
    x1 += ks2 + jnp.uint32(4)
    x0, x1 = rounds(x0, x1, ra)
    x0 += ks2
    x1 += ks0 + jnp.uint32(5)
    bits = x0 ^ x1

    fb = (bits >> jnp.uint32(9)) | jnp.uint32(0x3F800000)
    u_ref[...] = lax.bitcast_convert_type(fb, jnp.float32) - jnp.float32(1.0)


def _gen_uniform():
    block_cols = 4096
    return pl.pallas_call(
        functools.partial(_ugen_body, block_cols=block_cols),
        grid=(N // block_cols,),
        in_specs=[],
        out_specs=pl.BlockSpec((R, block_cols), lambda i: (0, i)),
        out_shape=jax.ShapeDtypeStruct((R, N), jnp.float32),
    )()


def _sel_body(x_ref, p_ref, u_ref, o_ref):
    x = x_ref[...]
    keep = u_ref[...] < (jnp.float32(1.0) - p_ref[...])
    o_ref[...] = jnp.where(keep, x, jnp.float32(0.0))


def _apply_mask(x, prob, u):
    block_cols = 8192
    return pl.pallas_call(
        _sel_body,
        grid=(N // block_cols,),
        in_specs=[
            pl.BlockSpec((R, block_cols), lambda i: (0, i)),
            pl.BlockSpec((R, block_cols), lambda i: (0, i)),
            pl.BlockSpec((R, block_cols), lambda i: (0, i)),
        ],
        out_specs=pl.BlockSpec((R, block_cols), lambda i: (0, i)),
        out_shape=jax.ShapeDtypeStruct((R, N), jnp.float32),
    )(x, prob, u)


def kernel(x):
    u = _gen_uniform()
    new_prob = _sc_topk_replace(x)
    return _apply_mask(x, new_prob, u)
